# Initial kernel scaffold; baseline (speedup 1.0000x reference)
import jax
import jax.numpy as jnp
from jax.experimental import pallas as pl


def kernel(rand_boxes_init, pseudo_scores, num_of_boxes_per_img):
    raise NotImplementedError("write your pallas kernel here")


# speedup vs baseline: 1144.8049x; 1144.8049x over previous
import numpy as np
import jax
import jax.numpy as jnp
from jax.experimental import pallas as pl
from jax.experimental.pallas import tpu as pltpu

H_IMG = 800.0
W_IMG = 1333.0
NMS_THR = 0.7
MIN_SCALE_RATE = 0.1
MIN_FINAL = 5
MAX_FINAL = 50
NUM_IMG = 4
NUM_INIT = 5000

_ROUNDS = MAX_FINAL - 1
_NPAD = 5120
_SLOT = 64
_BIGI = np.int32(2 ** 30)

_H_MIN = np.float32(H_IMG * MIN_SCALE_RATE)
_W_MIN = np.float32(W_IMG * MIN_SCALE_RATE)


def _nms_body(a_ref, b_ref, c_ref, d_ref, ps_ref, nb_ref,
              ox1_ref, oy1_ref, ox2_ref, oy2_ref, cnt_ref,
              sc_ref, x1_ref, y1_ref, x2_ref, y2_ref, ar_ref):
    a = a_ref[...]
    b = b_ref[...]
    c = c_ref[...]
    d = d_ref[...]
    x1 = jnp.minimum(a, c) * W_IMG
    x2 = jnp.maximum(a, c) * W_IMG
    y1 = jnp.minimum(b, d) * H_IMG
    y2 = jnp.maximum(b, d) * H_IMG
    bw = x2 - x1
    bh = y2 - y1
    col = jax.lax.broadcasted_iota(jnp.int32, (NUM_IMG, _NPAD), 1)
    mask = (bh > _H_MIN) & (bw > _W_MIN) & (col < NUM_INIT)
    x1_ref[...] = x1
    y1_ref[...] = y1
    x2_ref[...] = x2
    y2_ref[...] = y2
    ar_ref[...] = bw * bh
    sc_ref[...] = jnp.where(mask, ps_ref[...], -1.0)

    zero_f = jnp.zeros((NUM_IMG, _SLOT), jnp.float32)
    ox1_ref[...] = zero_f
    oy1_ref[...] = zero_f
    ox2_ref[...] = zero_f
    oy2_ref[...] = zero_f
    cnt_ref[...] = jnp.zeros((NUM_IMG, _SLOT), jnp.int32)

    nf = jnp.clip(nb_ref[...], MIN_FINAL, MAX_FINAL - 1)
    rcol = jax.lax.broadcasted_iota(jnp.int32, (NUM_IMG, _SLOT), 1)

    def round_body(k, _):
        sc = sc_ref[...]
        sm = jnp.max(sc, axis=1, keepdims=True)
        found = sm > -0.5
        cand = jnp.where(sc == sm, col, _BIGI)
        im = jnp.min(cand, axis=1, keepdims=True)
        sel = (col == im) & found
        x1m = jnp.sum(jnp.where(sel, x1_ref[...], 0.0), axis=1, keepdims=True)
        y1m = jnp.sum(jnp.where(sel, y1_ref[...], 0.0), axis=1, keepdims=True)
        x2m = jnp.sum(jnp.where(sel, x2_ref[...], 0.0), axis=1, keepdims=True)
        y2m = jnp.sum(jnp.where(sel, y2_ref[...], 0.0), axis=1, keepdims=True)
        am = jnp.sum(jnp.where(sel, ar_ref[...], 0.0), axis=1, keepdims=True)
        xx1 = jnp.maximum(x1m, x1_ref[...])
        yy1 = jnp.maximum(y1m, y1_ref[...])
        xx2 = jnp.minimum(x2m, x2_ref[...])
        yy2 = jnp.minimum(y2m, y2_ref[...])
        w = jnp.maximum(0.0, xx2 - xx1)
        h = jnp.maximum(0.0, yy2 - yy1)
        inter = w * h
        iou = inter / (am + ar_ref[...] - inter + 1e-9)
        sc_ref[...] = jnp.where(iou > NMS_THR, -1.0, sc)

        write = found & (k < nf)
        wmask = (rcol == k) & write
        ox1_ref[...] += jnp.where(wmask, x1m, 0.0)
        oy1_ref[...] += jnp.where(wmask, y1m, 0.0)
        ox2_ref[...] += jnp.where(wmask, x2m, 0.0)
        oy2_ref[...] += jnp.where(wmask, y2m, 0.0)
        cnt_ref[...] += jnp.where(write, 1, 0)
        return 0

    jax.lax.fori_loop(0, _ROUNDS, round_body, 0)


def kernel(rand_boxes_init, pseudo_scores, num_of_boxes_per_img):
    pad = _NPAD - NUM_INIT
    a = jnp.pad(rand_boxes_init[..., 0], ((0, 0), (0, pad)))
    b = jnp.pad(rand_boxes_init[..., 1], ((0, 0), (0, pad)))
    c = jnp.pad(rand_boxes_init[..., 2], ((0, 0), (0, pad)))
    d = jnp.pad(rand_boxes_init[..., 3], ((0, 0), (0, pad)))
    ps = jnp.pad(pseudo_scores, ((0, 0), (0, pad)))
    nb = num_of_boxes_per_img.reshape(NUM_IMG, 1)

    f_out = jax.ShapeDtypeStruct((NUM_IMG, _SLOT), jnp.float32)
    i_out = jax.ShapeDtypeStruct((NUM_IMG, _SLOT), jnp.int32)
    big = pltpu.VMEM((NUM_IMG, _NPAD), jnp.float32)
    ox1, oy1, ox2, oy2, cnt = pl.pallas_call(
        _nms_body,
        out_shape=(f_out, f_out, f_out, f_out, i_out),
        scratch_shapes=[big, big, big, big, big, big],
    )(a, b, c, d, ps, nb)

    out = jnp.stack([ox1[:, :MAX_FINAL], oy1[:, :MAX_FINAL],
                     ox2[:, :MAX_FINAL], oy2[:, :MAX_FINAL]], axis=-1)
    counts = cnt[:, 0]
    return out, counts
